# baseline (speedup 1.0000x reference)
import jax
import jax.numpy as jnp
from jax import lax
from jax.experimental import pallas as pl
from jax.experimental.pallas import tpu as pltpu
from jax.experimental.pallas import tpu_sc as plsc

BATCH = 4096
MAX_LEN = 200
WORD_DIM = 64
POS_DIM = 16
OUT_DIM = WORD_DIM + POS_DIM
NW = 32
BTILE = BATCH // NW
NSTEP = MAX_LEN // 2


def _out_slab(out_hbm, t, wid):
    return out_hbm.at[pl.ds(t * 10, 10), pl.ds(wid, 1), :]


def _sc_body(wt_hbm, pt_hbm, ww_hbm, wp_hbm, out_hbm,
             iw0, ip0, rw0, rp0, tb0,
             iw1, ip1, rw1, rp1, tb1,
             sg0, sw0, sg1, sw1):
    wid = lax.axis_index("s") * 2 + lax.axis_index("c")
    col0 = wid * BTILE
    iota = lax.iota(jnp.int32, 16)
    bvecs = [iota + bg * 16 for bg in range(8)]

    def fire_gathers(t, iw, ip, rw, rp, sg):
        pltpu.sync_copy(wt_hbm.at[t, pl.ds(col0, BTILE)], iw)
        pltpu.sync_copy(pt_hbm.at[t, pl.ds(col0, BTILE)], ip)
        return [pltpu.async_copy(ww_hbm.at[iw], rw, sg),
                pltpu.async_copy(wp_hbm.at[ip], rp, sg)]

    def transpose(rw, rp, tb):
        for c in range(OUT_DIM):
            src = rw if c < WORD_DIM else rp
            cvec = jnp.full((16,), c if c < WORD_DIM else c - WORD_DIM,
                            jnp.int32)
            for bg in range(8):
                v = plsc.load_gather(src, [bvecs[bg], cvec])
                flat = c * BTILE + bg * 16
                tb[flat // 1024, 0, pl.ds(flat % 1024, 16)] = v

    def body(i, carry):
        t0 = 2 * i
        t1 = 2 * i + 1
        g0 = fire_gathers(t0, iw0, ip0, rw0, rp0, sg0)
        g1 = fire_gathers(t1, iw1, ip1, rw1, rp1, sg1)

        @pl.when(i > 0)
        def _():
            pltpu.make_async_copy(tb0, _out_slab(out_hbm, t0 - 2, wid),
                                  sw0).wait()
            pltpu.make_async_copy(tb1, _out_slab(out_hbm, t1 - 2, wid),
                                  sw1).wait()

        for g in g0:
            g.wait()
        transpose(rw0, rp0, tb0)
        pltpu.async_copy(tb0, _out_slab(out_hbm, t0, wid), sw0)
        for g in g1:
            g.wait()
        transpose(rw1, rp1, tb1)
        pltpu.async_copy(tb1, _out_slab(out_hbm, t1, wid), sw1)
        return carry

    lax.fori_loop(0, NSTEP, body, 0)
    pltpu.make_async_copy(tb0, _out_slab(out_hbm, MAX_LEN - 2, wid),
                          sw0).wait()
    pltpu.make_async_copy(tb1, _out_slab(out_hbm, MAX_LEN - 1, wid),
                          sw1).wait()


def kernel(word, pos, W_word, W_pos):
    wt = word.T.astype(jnp.int32)
    pt = pos.T.astype(jnp.int32)
    mesh = plsc.VectorSubcoreMesh(core_axis_name="c", subcore_axis_name="s")
    set_types = [
        pltpu.VMEM((BTILE,), jnp.int32),
        pltpu.VMEM((BTILE,), jnp.int32),
        pltpu.VMEM((BTILE, WORD_DIM), jnp.float32),
        pltpu.VMEM((BTILE, POS_DIM), jnp.float32),
        pltpu.VMEM((10, 1, 1024), jnp.float32),
    ]
    out3 = pl.kernel(
        _sc_body,
        mesh=mesh,
        out_type=jax.ShapeDtypeStruct((MAX_LEN * 10, NW, 1024), jnp.float32),
        compiler_params=pltpu.CompilerParams(
            use_tc_tiling_on_sc=False, needs_layout_passes=False),
        scratch_types=set_types + set_types + [
            pltpu.SemaphoreType.DMA,
            pltpu.SemaphoreType.DMA,
            pltpu.SemaphoreType.DMA,
            pltpu.SemaphoreType.DMA,
        ],
    )(wt, pt, W_word, W_pos)
    x = out3.reshape(MAX_LEN, 10, NW, 8, BTILE)
    x = jnp.transpose(x, (2, 4, 0, 1, 3))
    return x.reshape(BATCH, MAX_LEN, OUT_DIM)


# speedup vs baseline: 1.2529x; 1.2529x over previous
import jax
import jax.numpy as jnp
from jax import lax
from jax.experimental import pallas as pl
from jax.experimental.pallas import tpu as pltpu
from jax.experimental.pallas import tpu_sc as plsc

BATCH = 4096
MAX_LEN = 200
WORD_DIM = 64
POS_DIM = 16
OUT_DIM = WORD_DIM + POS_DIM
NW = 32
BTILE = BATCH // NW
NSTEP = MAX_LEN // 2


def _out_slab(out_hbm, t, wid):
    return out_hbm.at[pl.ds(t * 10, 10), pl.ds(wid, 1), :]


def _sc_body(wt_hbm, pt_hbm, ww_hbm, wp_hbm, out_hbm,
             iw0, ip0, rw0, rp0, tb0,
             iw1, ip1, rw1, rp1, tb1,
             sg0, sw0, sg1, sw1):
    wid = lax.axis_index("s") * 2 + lax.axis_index("c")
    col0 = wid * BTILE
    iota = lax.iota(jnp.int32, 16)
    iota128 = iota * 128
    zvec = jnp.zeros((16,), jnp.int32)

    def fire_gathers(t, iw, ip, rw, rp, sg):
        pltpu.sync_copy(wt_hbm.at[t, pl.ds(col0, BTILE)], iw)
        pltpu.sync_copy(pt_hbm.at[t, pl.ds(col0, BTILE)], ip)
        return [pltpu.async_copy(ww_hbm.at[iw], rw, sg),
                pltpu.async_copy(wp_hbm.at[ip], rp, sg)]

    def transpose(rw, rp, tb):
        for b in range(BTILE):
            for blk in range(5):
                if blk < 4:
                    v = rw[b, pl.ds(blk * 16, 16)]
                else:
                    v = rp[b, pl.ds(0, 16)]
                f = iota128 + (blk * 16 * BTILE + b)
                q = lax.shift_right_logical(f, 10)
                r = lax.bitwise_and(f, 1023)
                plsc.store_scatter(tb, [q, zvec, r], v)

    def body(i, carry):
        t0 = 2 * i
        t1 = 2 * i + 1
        g0 = fire_gathers(t0, iw0, ip0, rw0, rp0, sg0)
        g1 = fire_gathers(t1, iw1, ip1, rw1, rp1, sg1)

        @pl.when(i > 0)
        def _():
            pltpu.make_async_copy(tb0, _out_slab(out_hbm, t0 - 2, wid),
                                  sw0).wait()
            pltpu.make_async_copy(tb1, _out_slab(out_hbm, t1 - 2, wid),
                                  sw1).wait()

        for g in g0:
            g.wait()
        transpose(rw0, rp0, tb0)
        pltpu.async_copy(tb0, _out_slab(out_hbm, t0, wid), sw0)
        for g in g1:
            g.wait()
        transpose(rw1, rp1, tb1)
        pltpu.async_copy(tb1, _out_slab(out_hbm, t1, wid), sw1)
        return carry

    lax.fori_loop(0, NSTEP, body, 0)
    pltpu.make_async_copy(tb0, _out_slab(out_hbm, MAX_LEN - 2, wid),
                          sw0).wait()
    pltpu.make_async_copy(tb1, _out_slab(out_hbm, MAX_LEN - 1, wid),
                          sw1).wait()


def kernel(word, pos, W_word, W_pos):
    wt = word.T.astype(jnp.int32)
    pt = pos.T.astype(jnp.int32)
    mesh = plsc.VectorSubcoreMesh(core_axis_name="c", subcore_axis_name="s")
    set_types = [
        pltpu.VMEM((BTILE,), jnp.int32),
        pltpu.VMEM((BTILE,), jnp.int32),
        pltpu.VMEM((BTILE, WORD_DIM), jnp.float32),
        pltpu.VMEM((BTILE, POS_DIM), jnp.float32),
        pltpu.VMEM((10, 1, 1024), jnp.float32),
    ]
    out3 = pl.kernel(
        _sc_body,
        mesh=mesh,
        out_type=jax.ShapeDtypeStruct((MAX_LEN * 10, NW, 1024), jnp.float32),
        compiler_params=pltpu.CompilerParams(
            use_tc_tiling_on_sc=False, needs_layout_passes=False),
        scratch_types=set_types + set_types + [
            pltpu.SemaphoreType.DMA,
            pltpu.SemaphoreType.DMA,
            pltpu.SemaphoreType.DMA,
            pltpu.SemaphoreType.DMA,
        ],
    )(wt, pt, W_word, W_pos)
    x = out3.reshape(MAX_LEN, 10, NW, 8, BTILE)
    x = jnp.transpose(x, (2, 4, 0, 1, 3))
    return x.reshape(BATCH, MAX_LEN, OUT_DIM)


# speedup vs baseline: 1.4828x; 1.1835x over previous
import jax
import jax.numpy as jnp
from jax import lax
from jax.experimental import pallas as pl
from jax.experimental.pallas import tpu as pltpu
from jax.experimental.pallas import tpu_sc as plsc

BATCH = 4096
MAX_LEN = 200
WORD_DIM = 64
POS_DIM = 16
OUT_DIM = WORD_DIM + POS_DIM
NW = 32
BTILE = BATCH // NW
NSTEP = MAX_LEN // 2


def _out_slab(out_hbm, t, wid):
    return out_hbm.at[pl.ds(t * 10, 10), pl.ds(wid, 1), :]


def _sc_body(wt_hbm, pt_hbm, ww_hbm, wp_hbm, out_hbm,
             iw0, ip0, rw0, rp0, tb0,
             iw1, ip1, rw1, rp1, tb1,
             sg0, sw0, sg1, sw1):
    wid = lax.axis_index("s") * 2 + lax.axis_index("c")
    col0 = wid * BTILE
    iota = lax.iota(jnp.int32, 16)
    zvec = jnp.zeros((16,), jnp.int32)
    cvecs = [iota + blk * 16 for blk in range(4)]
    fbases = [(iota + blk * 16) * BTILE for blk in range(5)]

    def fire_gathers(t, iw, ip, rw, rp, sg):
        pltpu.sync_copy(wt_hbm.at[t, pl.ds(col0, BTILE)], iw)
        pltpu.sync_copy(pt_hbm.at[t, pl.ds(col0, BTILE)], ip)
        return [pltpu.async_copy(ww_hbm.at[iw], rw, sg),
                pltpu.async_copy(wp_hbm.at[ip], rp, sg)]

    def transpose(rw, rp, tb):
        for b in range(BTILE):
            bb = lax.bitwise_and(iota + b, BTILE - 1)
            for blk in range(5):
                if blk < 4:
                    v = plsc.load_gather(rw, [bb, cvecs[blk]])
                else:
                    v = plsc.load_gather(rp, [bb, iota])
                f = fbases[blk] + bb
                q = lax.shift_right_logical(f, 10)
                r = lax.bitwise_and(f, 1023)
                plsc.store_scatter(tb, [q, zvec, r], v)

    def body(i, carry):
        t0 = 2 * i
        t1 = 2 * i + 1
        g0 = fire_gathers(t0, iw0, ip0, rw0, rp0, sg0)
        g1 = fire_gathers(t1, iw1, ip1, rw1, rp1, sg1)

        @pl.when(i > 0)
        def _():
            pltpu.make_async_copy(tb0, _out_slab(out_hbm, t0 - 2, wid),
                                  sw0).wait()
            pltpu.make_async_copy(tb1, _out_slab(out_hbm, t1 - 2, wid),
                                  sw1).wait()

        for g in g0:
            g.wait()
        transpose(rw0, rp0, tb0)
        pltpu.async_copy(tb0, _out_slab(out_hbm, t0, wid), sw0)
        for g in g1:
            g.wait()
        transpose(rw1, rp1, tb1)
        pltpu.async_copy(tb1, _out_slab(out_hbm, t1, wid), sw1)
        return carry

    lax.fori_loop(0, NSTEP, body, 0)
    pltpu.make_async_copy(tb0, _out_slab(out_hbm, MAX_LEN - 2, wid),
                          sw0).wait()
    pltpu.make_async_copy(tb1, _out_slab(out_hbm, MAX_LEN - 1, wid),
                          sw1).wait()


def kernel(word, pos, W_word, W_pos):
    wt = word.T.astype(jnp.int32)
    pt = pos.T.astype(jnp.int32)
    mesh = plsc.VectorSubcoreMesh(core_axis_name="c", subcore_axis_name="s")
    set_types = [
        pltpu.VMEM((BTILE,), jnp.int32),
        pltpu.VMEM((BTILE,), jnp.int32),
        pltpu.VMEM((BTILE, WORD_DIM), jnp.float32),
        pltpu.VMEM((BTILE, POS_DIM), jnp.float32),
        pltpu.VMEM((10, 1, 1024), jnp.float32),
    ]
    out3 = pl.kernel(
        _sc_body,
        mesh=mesh,
        out_type=jax.ShapeDtypeStruct((MAX_LEN * 10, NW, 1024), jnp.float32),
        compiler_params=pltpu.CompilerParams(
            use_tc_tiling_on_sc=False, needs_layout_passes=False),
        scratch_types=set_types + set_types + [
            pltpu.SemaphoreType.DMA,
            pltpu.SemaphoreType.DMA,
            pltpu.SemaphoreType.DMA,
            pltpu.SemaphoreType.DMA,
        ],
    )(wt, pt, W_word, W_pos)
    x = out3.reshape(MAX_LEN, 10, NW, 8, BTILE)
    x = jnp.transpose(x, (2, 4, 0, 1, 3))
    return x.reshape(BATCH, MAX_LEN, OUT_DIM)


# speedup vs baseline: 3.6815x; 2.4828x over previous
import jax
import jax.numpy as jnp
from jax import lax
from jax.experimental import pallas as pl
from jax.experimental.pallas import tpu as pltpu
from jax.experimental.pallas import tpu_sc as plsc

BATCH = 4096
MAX_LEN = 200
WORD_DIM = 64
POS_DIM = 16
OUT_DIM = WORD_DIM + POS_DIM
NW = 32
BTILE = BATCH // NW
NSTEP = MAX_LEN // 2


def _out_slab(out_hbm, t, wid):
    return out_hbm.at[pl.ds(t * 10, 10), pl.ds(wid, 1), :]


def _sc_body(wt_hbm, pt_hbm, ww_hbm, wp_hbm, out_hbm,
             iw0, ip0, rw0, rp0, tb0,
             iw1, ip1, rw1, rp1, tb1,
             sg0, sw0, sg1, sw1):
    wid = lax.axis_index("s") * 2 + lax.axis_index("c")
    col0 = wid * BTILE
    iota = lax.iota(jnp.int32, 16)
    zvec = jnp.zeros((16,), jnp.int32)
    cvecs = [iota + blk * 16 for blk in range(4)]
    fbases = [(iota + blk * 16) * BTILE for blk in range(5)]

    def fire_gathers(t, iw, ip, rw, rp, sg):
        pltpu.sync_copy(wt_hbm.at[t, pl.ds(col0, BTILE)], iw)
        pltpu.sync_copy(pt_hbm.at[t, pl.ds(col0, BTILE)], ip)
        return [pltpu.async_copy(ww_hbm.at[iw], rw, sg),
                pltpu.async_copy(wp_hbm.at[ip], rp, sg)]

    def transpose(rw, rp, tb):
        @plsc.parallel_loop(0, BTILE, unroll=8)
        def _(b):
            bb = lax.bitwise_and(iota + b, BTILE - 1)
            for blk in range(5):
                if blk < 4:
                    v = plsc.load_gather(rw, [bb, cvecs[blk]])
                else:
                    v = plsc.load_gather(rp, [bb, iota])
                f = fbases[blk] + bb
                q = lax.shift_right_logical(f, 10)
                r = lax.bitwise_and(f, 1023)
                plsc.store_scatter(tb, [q, zvec, r], v)

    def body(i, carry):
        t0 = 2 * i
        t1 = 2 * i + 1
        g0 = fire_gathers(t0, iw0, ip0, rw0, rp0, sg0)
        g1 = fire_gathers(t1, iw1, ip1, rw1, rp1, sg1)

        @pl.when(i > 0)
        def _():
            pltpu.make_async_copy(tb0, _out_slab(out_hbm, t0 - 2, wid),
                                  sw0).wait()
            pltpu.make_async_copy(tb1, _out_slab(out_hbm, t1 - 2, wid),
                                  sw1).wait()

        for g in g0:
            g.wait()
        transpose(rw0, rp0, tb0)
        pltpu.async_copy(tb0, _out_slab(out_hbm, t0, wid), sw0)
        for g in g1:
            g.wait()
        transpose(rw1, rp1, tb1)
        pltpu.async_copy(tb1, _out_slab(out_hbm, t1, wid), sw1)
        return carry

    lax.fori_loop(0, NSTEP, body, 0)
    pltpu.make_async_copy(tb0, _out_slab(out_hbm, MAX_LEN - 2, wid),
                          sw0).wait()
    pltpu.make_async_copy(tb1, _out_slab(out_hbm, MAX_LEN - 1, wid),
                          sw1).wait()


def kernel(word, pos, W_word, W_pos):
    wt = word.T.astype(jnp.int32)
    pt = pos.T.astype(jnp.int32)
    mesh = plsc.VectorSubcoreMesh(core_axis_name="c", subcore_axis_name="s")
    set_types = [
        pltpu.VMEM((BTILE,), jnp.int32),
        pltpu.VMEM((BTILE,), jnp.int32),
        pltpu.VMEM((BTILE, WORD_DIM), jnp.float32),
        pltpu.VMEM((BTILE, POS_DIM), jnp.float32),
        pltpu.VMEM((10, 1, 1024), jnp.float32),
    ]
    out3 = pl.kernel(
        _sc_body,
        mesh=mesh,
        out_type=jax.ShapeDtypeStruct((MAX_LEN * 10, NW, 1024), jnp.float32),
        compiler_params=pltpu.CompilerParams(
            use_tc_tiling_on_sc=False, needs_layout_passes=False),
        scratch_types=set_types + set_types + [
            pltpu.SemaphoreType.DMA,
            pltpu.SemaphoreType.DMA,
            pltpu.SemaphoreType.DMA,
            pltpu.SemaphoreType.DMA,
        ],
    )(wt, pt, W_word, W_pos)
    x = out3.reshape(MAX_LEN, 10, NW, 8, BTILE)
    x = jnp.transpose(x, (2, 4, 0, 1, 3))
    return x.reshape(BATCH, MAX_LEN, OUT_DIM)


# speedup vs baseline: 5.7767x; 1.5691x over previous
import jax
import jax.numpy as jnp
from jax import lax
from jax.experimental import pallas as pl
from jax.experimental.pallas import tpu as pltpu
from jax.experimental.pallas import tpu_sc as plsc

BATCH = 4096
MAX_LEN = 200
WORD_DIM = 64
POS_DIM = 16
OUT_DIM = WORD_DIM + POS_DIM
NW = 32
BTILE = BATCH // NW
NSET = 4
NITER = MAX_LEN // NSET


def _out_slab(out_hbm, t, wid):
    return out_hbm.at[pl.ds(t * 10, 10), pl.ds(wid, 1), :]


def _sc_body(wt_hbm, pt_hbm, ww_hbm, wp_hbm, out_hbm,
             iw_all, ip_all,
             rw0, rp0, rw1, rp1, rw2, rp2, rw3, rp3,
             tb0, tb1,
             sg0, sg1, sg2, sg3, sw0, sw1):
    rws = (rw0, rw1, rw2, rw3)
    rps = (rp0, rp1, rp2, rp3)
    sgs = (sg0, sg1, sg2, sg3)
    tbs = (tb0, tb1)
    sws = (sw0, sw1)
    wid = lax.axis_index("s") * 2 + lax.axis_index("c")
    col0 = wid * BTILE
    iota = lax.iota(jnp.int32, 16)
    zvec = jnp.zeros((16,), jnp.int32)
    cvecs = [iota + blk * 16 for blk in range(4)]
    fbases = [(iota + blk * 16) * BTILE for blk in range(5)]

    pltpu.sync_copy(wt_hbm.at[:, pl.ds(col0, BTILE)], iw_all)
    pltpu.sync_copy(pt_hbm.at[:, pl.ds(col0, BTILE)], ip_all)

    def fire_g(s, t):
        pltpu.async_copy(ww_hbm.at[iw_all.at[t]], rws[s], sgs[s])
        pltpu.async_copy(wp_hbm.at[ip_all.at[t]], rps[s], sgs[s])

    def wait_g(s, t):
        pltpu.make_async_copy(ww_hbm.at[iw_all.at[t]], rws[s], sgs[s]).wait()
        pltpu.make_async_copy(wp_hbm.at[ip_all.at[t]], rps[s], sgs[s]).wait()

    def wait_w(k, t):
        pltpu.make_async_copy(tbs[k], _out_slab(out_hbm, t, wid),
                              sws[k]).wait()

    def transpose(rw, rp, tb):
        @plsc.parallel_loop(0, BTILE, unroll=8)
        def _(b):
            bb = lax.bitwise_and(iota + b, BTILE - 1)
            for blk in range(5):
                if blk < 4:
                    v = plsc.load_gather(rw, [bb, cvecs[blk]])
                else:
                    v = plsc.load_gather(rp, [bb, iota])
                f = fbases[blk] + bb
                q = lax.shift_right_logical(f, 10)
                r = lax.bitwise_and(f, 1023)
                plsc.store_scatter(tb, [q, zvec, r], v)

    for s in range(NSET):
        fire_g(s, s)

    def body(i, carry):
        t0 = NSET * i
        for s in range(NSET):
            t = t0 + s
            k = s % 2
            wait_g(s, t)
            if s < 2:
                @pl.when(i > 0)
                def _():
                    wait_w(k, t)
            else:
                wait_w(k, t)
            transpose(rws[s], rps[s], tbs[k])
            pltpu.async_copy(tbs[k], _out_slab(out_hbm, t, wid), sws[k])

            @pl.when(i < NITER - 1)
            def _():
                fire_g(s, t + NSET)
        return carry

    lax.fori_loop(0, NITER, body, 0)
    wait_w(0, MAX_LEN - 2)
    wait_w(1, MAX_LEN - 1)


def kernel(word, pos, W_word, W_pos):
    wt = word.T.astype(jnp.int32)
    pt = pos.T.astype(jnp.int32)
    mesh = plsc.VectorSubcoreMesh(core_axis_name="c", subcore_axis_name="s")
    gather_set = [
        pltpu.VMEM((BTILE, WORD_DIM), jnp.float32),
        pltpu.VMEM((BTILE, POS_DIM), jnp.float32),
    ]
    out3 = pl.kernel(
        _sc_body,
        mesh=mesh,
        out_type=jax.ShapeDtypeStruct((MAX_LEN * 10, NW, 1024), jnp.float32),
        compiler_params=pltpu.CompilerParams(
            use_tc_tiling_on_sc=False, needs_layout_passes=False),
        scratch_types=[
            pltpu.VMEM((MAX_LEN, BTILE), jnp.int32),
            pltpu.VMEM((MAX_LEN, BTILE), jnp.int32),
        ] + gather_set * NSET + [
            pltpu.VMEM((10, 1, 1024), jnp.float32),
            pltpu.VMEM((10, 1, 1024), jnp.float32),
        ] + [pltpu.SemaphoreType.DMA] * (NSET + 2),
    )(wt, pt, W_word, W_pos)
    x = out3.reshape(MAX_LEN, 10, NW, 8, BTILE)
    x = jnp.transpose(x, (2, 4, 0, 1, 3))
    return x.reshape(BATCH, MAX_LEN, OUT_DIM)
